# baseline (speedup 1.0000x reference)
import functools
import jax
import jax.numpy as jnp
from jax import lax
from jax.experimental import pallas as pl
from jax.experimental.pallas import tpu as pltpu
from jax.experimental.pallas import tpu_sc as plsc

N = 10000
E = 320000
D = 128

NP = 10240
NC = 2
NS = 16
NW = NC * NS
EPW = E // NW
CH = 80
NCHUNK = EPW // CH
ROWS_PT = NP // NS
ZROWS = 64

_mesh = plsc.VectorSubcoreMesh(core_axis_name="c", subcore_axis_name="s",
                               num_cores=NC, num_subcores=NS)


def _agg_body(g_hbm, src_hbm, dst_hbm, zeros_hbm, out_hbm,
              src_v, dst_v, rows_v, acc, gsem, ssem):
    c = lax.axis_index("c")
    s = lax.axis_index("s")
    wid = s * NC + c
    ebase = wid * EPW
    rbase = s * ROWS_PT

    @pl.when(c == 0)
    def _():
        pltpu.sync_copy(g_hbm.at[pl.ds(rbase, ROWS_PT)],
                        acc.at[pl.ds(rbase, ROWS_PT)])

    @pl.when(c != 0)
    def _():
        def zcopy(t, _):
            pltpu.sync_copy(zeros_hbm,
                            acc.at[pl.ds(rbase + t * ZROWS, ZROWS)])
            return 0

        lax.fori_loop(0, ROWS_PT // ZROWS, zcopy, 0)

    plsc.subcore_barrier()

    pltpu.sync_copy(src_hbm.at[pl.ds(ebase, CH)], src_v.at[0])
    pltpu.sync_copy(dst_hbm.at[pl.ds(ebase, CH)], dst_v.at[0])
    pltpu.async_copy(g_hbm.at[src_v.at[0]], rows_v.at[0], gsem)

    def chunk(j, _):
        p = lax.rem(j, 2)
        q = lax.rem(j + 1, 2)

        @pl.when(j > 0)
        def _():
            pltpu.make_async_copy(rows_v.at[q], acc.at[dst_v.at[q]],
                                  ssem).wait()

        @pl.when(j < NCHUNK - 1)
        def _():
            pltpu.sync_copy(src_hbm.at[pl.ds(ebase + (j + 1) * CH, CH)],
                            src_v.at[q])
            pltpu.sync_copy(dst_hbm.at[pl.ds(ebase + (j + 1) * CH, CH)],
                            dst_v.at[q])
            pltpu.async_copy(g_hbm.at[src_v.at[q]], rows_v.at[q], gsem)

        pltpu.make_async_copy(g_hbm.at[src_v.at[p]], rows_v.at[p],
                              gsem).wait()
        pltpu.async_copy(rows_v.at[p], acc.at[dst_v.at[p]], ssem,
                         add=True)
        return 0

    lax.fori_loop(0, NCHUNK, chunk, 0)
    lp = (NCHUNK - 1) % 2
    pltpu.make_async_copy(rows_v.at[lp], acc.at[dst_v.at[lp]],
                          ssem).wait()

    plsc.subcore_barrier()
    pltpu.sync_copy(acc.at[pl.ds(rbase, ROWS_PT)],
                    out_hbm.at[c, pl.ds(rbase, ROWS_PT)])


_agg_call = pl.kernel(
    _agg_body,
    out_type=jax.ShapeDtypeStruct((NC, NP, D), jnp.float32),
    mesh=_mesh,
    scratch_types=[
        pltpu.VMEM((2, CH), jnp.int32),
        pltpu.VMEM((2, CH), jnp.int32),
        pltpu.VMEM((2, CH, D), jnp.float32),
        pltpu.VMEM_SHARED((NP, D), jnp.float32),
        pltpu.SemaphoreType.DMA,
        pltpu.SemaphoreType.DMA,
    ],
)


def _mm0_body(x_ref, w_ref, h_ref):
    h_ref[...] = jnp.dot(x_ref[...], w_ref[...],
                         preferred_element_type=jnp.float32)


def _scale0_body(h_ref, degp_ref, g_ref, dinv_ref):
    d = degp_ref[0, pl.ds(0, N), :] + degp_ref[1, pl.ds(0, N), :]
    deg = jnp.max(d, axis=1, keepdims=True)
    dinv = lax.rsqrt(deg)
    dinv_ref[...] = dinv
    g_ref[pl.ds(0, N), :] = h_ref[...] * dinv
    g_ref[pl.ds(N, NP - N), :] = jnp.zeros((NP - N, D), jnp.float32)


def _dense1_body(q_ref, dinv_ref, b0_ref, w1_ref, g_ref):
    dinv = dinv_ref[...]
    qsum = q_ref[0, pl.ds(0, N), :] + q_ref[1, pl.ds(0, N), :]
    t = qsum * dinv + b0_ref[...]
    a = jnp.where(t > 0, t, 0.01 * t)
    h = jnp.dot(a, w1_ref[...], preferred_element_type=jnp.float32)
    g_ref[pl.ds(0, N), :] = h * dinv
    g_ref[pl.ds(N, NP - N), :] = jnp.zeros((NP - N, D), jnp.float32)


def _dense2_body(r_ref, dinv_ref, b1_ref, out_ref):
    rsum = r_ref[0, pl.ds(0, N), :] + r_ref[1, pl.ds(0, N), :]
    out_ref[...] = rsum * dinv_ref[...] + b1_ref[...]


@jax.jit
def kernel(x, edge_index, W0, b0, W1, b1):
    src = edge_index[0]
    dst = edge_index[1]

    zeros_blk = jnp.zeros((ZROWS, D), jnp.float32)
    ones_pad = jnp.concatenate(
        [jnp.ones((N, D), jnp.float32), jnp.zeros((NP - N, D), jnp.float32)])

    degp = _agg_call(ones_pad, src, dst, zeros_blk)

    h0 = pl.pallas_call(
        _mm0_body,
        out_shape=jax.ShapeDtypeStruct((N, D), jnp.float32),
    )(x, W0)

    g0, dinv = pl.pallas_call(
        _scale0_body,
        out_shape=[
            jax.ShapeDtypeStruct((NP, D), jnp.float32),
            jax.ShapeDtypeStruct((N, 1), jnp.float32),
        ],
    )(h0, degp)

    q = _agg_call(g0, src, dst, zeros_blk)

    g1 = pl.pallas_call(
        _dense1_body,
        out_shape=jax.ShapeDtypeStruct((NP, D), jnp.float32),
    )(q, dinv, b0.reshape(1, D), W1)

    r = _agg_call(g1, src, dst, zeros_blk)

    out = pl.pallas_call(
        _dense2_body,
        out_shape=jax.ShapeDtypeStruct((N, D), jnp.float32),
    )(r, dinv, b1.reshape(1, D))

    return out


# speedup vs baseline: 1.2174x; 1.2174x over previous
import functools
import jax
import jax.numpy as jnp
from jax import lax
from jax.experimental import pallas as pl
from jax.experimental.pallas import tpu as pltpu
from jax.experimental.pallas import tpu_sc as plsc

N = 10000
E = 320000
D = 128

NP = 10240
NC = 2
NS = 16
NW = NC * NS
EPW = E // NW
CH = 80
NCHUNK = EPW // CH
ROWS_PT = NP // NS
ZROWS = 64
DEG_W = 16

_mesh = plsc.VectorSubcoreMesh(core_axis_name="c", subcore_axis_name="s",
                               num_cores=NC, num_subcores=NS)


def _agg_body(g_hbm, src_hbm, dst_hbm, zeros_hbm, out_hbm,
              src_v, dst_v, rows_v, acc, gsem, ssem):
    c = lax.axis_index("c")
    s = lax.axis_index("s")
    wid = s * NC + c
    ebase = wid * EPW
    rbase = s * ROWS_PT

    @pl.when(c == 0)
    def _():
        pltpu.sync_copy(g_hbm.at[pl.ds(rbase, ROWS_PT)],
                        acc.at[pl.ds(rbase, ROWS_PT)])

    @pl.when(c != 0)
    def _():
        def zcopy(t, _):
            pltpu.sync_copy(zeros_hbm,
                            acc.at[pl.ds(rbase + t * ZROWS, ZROWS)])
            return 0

        lax.fori_loop(0, ROWS_PT // ZROWS, zcopy, 0)

    plsc.subcore_barrier()

    pltpu.sync_copy(src_hbm.at[pl.ds(ebase, CH)], src_v.at[0])
    pltpu.sync_copy(dst_hbm.at[pl.ds(ebase, CH)], dst_v.at[0])
    pltpu.async_copy(g_hbm.at[src_v.at[0]], rows_v.at[0], gsem)

    def chunk(j, _):
        p = lax.rem(j, 2)
        q = lax.rem(j + 1, 2)

        @pl.when(j > 0)
        def _():
            pltpu.make_async_copy(rows_v.at[q], acc.at[dst_v.at[q]],
                                  ssem).wait()

        @pl.when(j < NCHUNK - 1)
        def _():
            pltpu.sync_copy(src_hbm.at[pl.ds(ebase + (j + 1) * CH, CH)],
                            src_v.at[q])
            pltpu.sync_copy(dst_hbm.at[pl.ds(ebase + (j + 1) * CH, CH)],
                            dst_v.at[q])
            pltpu.async_copy(g_hbm.at[src_v.at[q]], rows_v.at[q], gsem)

        pltpu.make_async_copy(g_hbm.at[src_v.at[p]], rows_v.at[p],
                              gsem).wait()
        pltpu.async_copy(rows_v.at[p], acc.at[dst_v.at[p]], ssem,
                         add=True)
        return 0

    lax.fori_loop(0, NCHUNK, chunk, 0)
    lp = (NCHUNK - 1) % 2
    pltpu.make_async_copy(rows_v.at[lp], acc.at[dst_v.at[lp]],
                          ssem).wait()

    plsc.subcore_barrier()
    pltpu.sync_copy(acc.at[pl.ds(rbase, ROWS_PT)],
                    out_hbm.at[c, pl.ds(rbase, ROWS_PT)])


_agg_call = pl.kernel(
    _agg_body,
    out_type=jax.ShapeDtypeStruct((NC, NP, D), jnp.float32),
    mesh=_mesh,
    scratch_types=[
        pltpu.VMEM((2, CH), jnp.int32),
        pltpu.VMEM((2, CH), jnp.int32),
        pltpu.VMEM((2, CH, D), jnp.float32),
        pltpu.VMEM_SHARED((NP, D), jnp.float32),
        pltpu.SemaphoreType.DMA,
        pltpu.SemaphoreType.DMA,
    ],
)


def _deg_body(dst_hbm, out_hbm, dst_v, ones_v, tcol_v, wide_v, acc, ssem):
    c = lax.axis_index("c")
    s = lax.axis_index("s")
    wid = s * NC + c
    ebase = wid * EPW
    rbase = s * ROWS_PT

    def fill_ones(i, _):
        ones_v[i, :] = jnp.ones((DEG_W,), jnp.float32)
        return 0

    lax.fori_loop(0, CH, fill_ones, 0)

    @pl.when(c == 0)
    def _():
        def fill_init(i, _):
            tcol_v[i, :] = jnp.ones((DEG_W,), jnp.float32)
            return 0
        lax.fori_loop(0, ZROWS, fill_init, 0)

    @pl.when(c != 0)
    def _():
        def fill_init(i, _):
            tcol_v[i, :] = jnp.zeros((DEG_W,), jnp.float32)
            return 0
        lax.fori_loop(0, ZROWS, fill_init, 0)

    def icopy(t, _):
        pltpu.sync_copy(tcol_v, acc.at[pl.ds(rbase + t * ZROWS, ZROWS)])
        return 0

    lax.fori_loop(0, ROWS_PT // ZROWS, icopy, 0)
    plsc.subcore_barrier()

    pltpu.sync_copy(dst_hbm.at[pl.ds(ebase, CH)], dst_v.at[0])

    def chunk(j, _):
        p = lax.rem(j, 2)
        q = lax.rem(j + 1, 2)

        @pl.when(j > 0)
        def _():
            pltpu.make_async_copy(ones_v, acc.at[dst_v.at[q]], ssem).wait()

        @pl.when(j < NCHUNK - 1)
        def _():
            pltpu.sync_copy(dst_hbm.at[pl.ds(ebase + (j + 1) * CH, CH)],
                            dst_v.at[q])

        pltpu.async_copy(ones_v, acc.at[dst_v.at[p]], ssem, add=True)
        return 0

    lax.fori_loop(0, NCHUNK, chunk, 0)
    lp = (NCHUNK - 1) % 2
    pltpu.make_async_copy(ones_v, acc.at[dst_v.at[lp]], ssem).wait()
    plsc.subcore_barrier()

    def wblock(t, _):
        pltpu.sync_copy(acc.at[pl.ds(rbase + t * ZROWS, ZROWS)], tcol_v)

        def repack(i, _):
            row = tcol_v[i, :]
            for k in range(D // DEG_W):
                wide_v[i, pl.ds(k * DEG_W, DEG_W)] = row
            return 0

        lax.fori_loop(0, ZROWS, repack, 0)
        pltpu.sync_copy(wide_v,
                        out_hbm.at[c, pl.ds(rbase + t * ZROWS, ZROWS)])
        return 0

    lax.fori_loop(0, ROWS_PT // ZROWS, wblock, 0)


_deg_call = pl.kernel(
    _deg_body,
    out_type=jax.ShapeDtypeStruct((NC, NP, D), jnp.float32),
    mesh=_mesh,
    scratch_types=[
        pltpu.VMEM((2, CH), jnp.int32),
        pltpu.VMEM((CH, DEG_W), jnp.float32),
        pltpu.VMEM((ZROWS, DEG_W), jnp.float32),
        pltpu.VMEM((ZROWS, D), jnp.float32),
        pltpu.VMEM_SHARED((NP, DEG_W), jnp.float32),
        pltpu.SemaphoreType.DMA,
    ],
    compiler_params=pltpu.CompilerParams(use_tc_tiling_on_sc=False),
)


def _mm0_body(x_ref, w_ref, h_ref):
    h_ref[...] = jnp.dot(x_ref[...], w_ref[...],
                         preferred_element_type=jnp.float32)


def _scale0_body(h_ref, degp_ref, g_ref, dinv_ref):
    d = degp_ref[0, pl.ds(0, N), :] + degp_ref[1, pl.ds(0, N), :]
    deg = jnp.max(d, axis=1, keepdims=True)
    dinv = lax.rsqrt(deg)
    dinv_ref[...] = dinv
    g_ref[pl.ds(0, N), :] = h_ref[...] * dinv
    g_ref[pl.ds(N, NP - N), :] = jnp.zeros((NP - N, D), jnp.float32)


def _dense1_body(q_ref, dinv_ref, b0_ref, w1_ref, g_ref):
    dinv = dinv_ref[...]
    qsum = q_ref[0, pl.ds(0, N), :] + q_ref[1, pl.ds(0, N), :]
    t = qsum * dinv + b0_ref[...]
    a = jnp.where(t > 0, t, 0.01 * t)
    h = jnp.dot(a, w1_ref[...], preferred_element_type=jnp.float32)
    g_ref[pl.ds(0, N), :] = h * dinv
    g_ref[pl.ds(N, NP - N), :] = jnp.zeros((NP - N, D), jnp.float32)


def _dense2_body(r_ref, dinv_ref, b1_ref, out_ref):
    rsum = r_ref[0, pl.ds(0, N), :] + r_ref[1, pl.ds(0, N), :]
    out_ref[...] = rsum * dinv_ref[...] + b1_ref[...]


@jax.jit
def kernel(x, edge_index, W0, b0, W1, b1):
    src = edge_index[0]
    dst = edge_index[1]

    zeros_blk = jnp.zeros((ZROWS, D), jnp.float32)

    degp = _deg_call(dst)

    h0 = pl.pallas_call(
        _mm0_body,
        out_shape=jax.ShapeDtypeStruct((N, D), jnp.float32),
    )(x, W0)

    g0, dinv = pl.pallas_call(
        _scale0_body,
        out_shape=[
            jax.ShapeDtypeStruct((NP, D), jnp.float32),
            jax.ShapeDtypeStruct((N, 1), jnp.float32),
        ],
    )(h0, degp)

    q = _agg_call(g0, src, dst, zeros_blk)

    g1 = pl.pallas_call(
        _dense1_body,
        out_shape=jax.ShapeDtypeStruct((NP, D), jnp.float32),
    )(q, dinv, b0.reshape(1, D), W1)

    r = _agg_call(g1, src, dst, zeros_blk)

    out = pl.pallas_call(
        _dense2_body,
        out_shape=jax.ShapeDtypeStruct((N, D), jnp.float32),
    )(r, dinv, b1.reshape(1, D))

    return out


# speedup vs baseline: 1.3465x; 1.1061x over previous
import functools
import jax
import jax.numpy as jnp
from jax import lax
from jax.experimental import pallas as pl
from jax.experimental.pallas import tpu as pltpu
from jax.experimental.pallas import tpu_sc as plsc

N = 10000
E = 320000
D = 128

NP = 10240
NC = 2
NS = 16
NW = NC * NS
EPW = E // NW
CH = 80
NCHUNK = EPW // CH
ROWS_PT = NP // NS
ZROWS = 32
DEG_W = 16

_mesh = plsc.VectorSubcoreMesh(core_axis_name="c", subcore_axis_name="s",
                               num_cores=NC, num_subcores=NS)


def _agg_body(g_hbm, src_hbm, dst_hbm, zeros_hbm, out_hbm,
              src_v, dst_v, rows_v, acc, gsem, ssem):
    c = lax.axis_index("c")
    s = lax.axis_index("s")
    wid = s * NC + c
    ebase = wid * EPW
    rbase = s * ROWS_PT

    @pl.when(c == 0)
    def _():
        pltpu.sync_copy(g_hbm.at[pl.ds(rbase, ROWS_PT)],
                        acc.at[pl.ds(rbase, ROWS_PT)])

    @pl.when(c != 0)
    def _():
        def zcopy(t, _):
            pltpu.sync_copy(zeros_hbm,
                            acc.at[pl.ds(rbase + t * ZROWS, ZROWS)])
            return 0

        lax.fori_loop(0, ROWS_PT // ZROWS, zcopy, 0)

    plsc.subcore_barrier()

    pltpu.sync_copy(src_hbm.at[pl.ds(ebase, CH)], src_v.at[0])
    pltpu.sync_copy(dst_hbm.at[pl.ds(ebase, CH)], dst_v.at[0])
    pltpu.async_copy(g_hbm.at[src_v.at[0]], rows_v.at[0], gsem)

    def chunk(j, _):
        p = lax.rem(j, 3)
        q = lax.rem(j + 1, 3)

        @pl.when(j >= 2)
        def _():
            pltpu.make_async_copy(rows_v.at[q], acc.at[dst_v.at[q]],
                                  ssem).wait()

        @pl.when(j < NCHUNK - 1)
        def _():
            pltpu.sync_copy(src_hbm.at[pl.ds(ebase + (j + 1) * CH, CH)],
                            src_v.at[q])
            pltpu.sync_copy(dst_hbm.at[pl.ds(ebase + (j + 1) * CH, CH)],
                            dst_v.at[q])
            pltpu.async_copy(g_hbm.at[src_v.at[q]], rows_v.at[q], gsem)

        pltpu.make_async_copy(g_hbm.at[src_v.at[p]], rows_v.at[p],
                              gsem).wait()
        pltpu.async_copy(rows_v.at[p], acc.at[dst_v.at[p]], ssem,
                         add=True)
        return 0

    lax.fori_loop(0, NCHUNK, chunk, 0)
    for t in (NCHUNK - 2, NCHUNK - 1):
        tp = t % 3
        pltpu.make_async_copy(rows_v.at[tp], acc.at[dst_v.at[tp]],
                              ssem).wait()

    plsc.subcore_barrier()
    pltpu.sync_copy(acc.at[pl.ds(rbase, ROWS_PT)],
                    out_hbm.at[c, pl.ds(rbase, ROWS_PT)])


_agg_call = pl.kernel(
    _agg_body,
    out_type=jax.ShapeDtypeStruct((NC, NP, D), jnp.float32),
    mesh=_mesh,
    scratch_types=[
        pltpu.VMEM((3, CH), jnp.int32),
        pltpu.VMEM((3, CH), jnp.int32),
        pltpu.VMEM((3, CH, D), jnp.float32),
        pltpu.VMEM_SHARED((NP, D), jnp.float32),
        pltpu.SemaphoreType.DMA,
        pltpu.SemaphoreType.DMA,
    ],
)


def _deg_body(dst_hbm, out_hbm, dst_v, ones_v, tcol_v, wide_v, acc, ssem):
    c = lax.axis_index("c")
    s = lax.axis_index("s")
    wid = s * NC + c
    ebase = wid * EPW
    rbase = s * ROWS_PT

    def fill_ones(i, _):
        ones_v[i, :] = jnp.ones((DEG_W,), jnp.float32)
        return 0

    lax.fori_loop(0, CH, fill_ones, 0)

    @pl.when(c == 0)
    def _():
        def fill_init(i, _):
            tcol_v[i, :] = jnp.ones((DEG_W,), jnp.float32)
            return 0
        lax.fori_loop(0, ZROWS, fill_init, 0)

    @pl.when(c != 0)
    def _():
        def fill_init(i, _):
            tcol_v[i, :] = jnp.zeros((DEG_W,), jnp.float32)
            return 0
        lax.fori_loop(0, ZROWS, fill_init, 0)

    def icopy(t, _):
        pltpu.sync_copy(tcol_v, acc.at[pl.ds(rbase + t * ZROWS, ZROWS)])
        return 0

    lax.fori_loop(0, ROWS_PT // ZROWS, icopy, 0)
    plsc.subcore_barrier()

    pltpu.sync_copy(dst_hbm.at[pl.ds(ebase, CH)], dst_v.at[0])

    def chunk(j, _):
        p = lax.rem(j, 2)
        q = lax.rem(j + 1, 2)

        @pl.when(j > 0)
        def _():
            pltpu.make_async_copy(ones_v, acc.at[dst_v.at[q]], ssem).wait()

        @pl.when(j < NCHUNK - 1)
        def _():
            pltpu.sync_copy(dst_hbm.at[pl.ds(ebase + (j + 1) * CH, CH)],
                            dst_v.at[q])

        pltpu.async_copy(ones_v, acc.at[dst_v.at[p]], ssem, add=True)
        return 0

    lax.fori_loop(0, NCHUNK, chunk, 0)
    lp = (NCHUNK - 1) % 2
    pltpu.make_async_copy(ones_v, acc.at[dst_v.at[lp]], ssem).wait()
    plsc.subcore_barrier()

    def wblock(t, _):
        pltpu.sync_copy(acc.at[pl.ds(rbase + t * ZROWS, ZROWS)], tcol_v)

        def repack(i, _):
            row = tcol_v[i, :]
            for k in range(D // DEG_W):
                wide_v[i, pl.ds(k * DEG_W, DEG_W)] = row
            return 0

        lax.fori_loop(0, ZROWS, repack, 0)
        pltpu.sync_copy(wide_v,
                        out_hbm.at[c, pl.ds(rbase + t * ZROWS, ZROWS)])
        return 0

    lax.fori_loop(0, ROWS_PT // ZROWS, wblock, 0)


_deg_call = pl.kernel(
    _deg_body,
    out_type=jax.ShapeDtypeStruct((NC, NP, D), jnp.float32),
    mesh=_mesh,
    scratch_types=[
        pltpu.VMEM((2, CH), jnp.int32),
        pltpu.VMEM((CH, DEG_W), jnp.float32),
        pltpu.VMEM((ZROWS, DEG_W), jnp.float32),
        pltpu.VMEM((ZROWS, D), jnp.float32),
        pltpu.VMEM_SHARED((NP, DEG_W), jnp.float32),
        pltpu.SemaphoreType.DMA,
    ],
    compiler_params=pltpu.CompilerParams(use_tc_tiling_on_sc=False),
)


def _mm0_body(x_ref, w_ref, h_ref):
    h_ref[...] = jnp.dot(x_ref[...], w_ref[...],
                         preferred_element_type=jnp.float32)


def _scale0_body(h_ref, degp_ref, g_ref, dinv_ref):
    d = degp_ref[0, pl.ds(0, N), :] + degp_ref[1, pl.ds(0, N), :]
    deg = jnp.max(d, axis=1, keepdims=True)
    dinv = lax.rsqrt(deg)
    dinv_ref[...] = dinv
    g_ref[pl.ds(0, N), :] = h_ref[...] * dinv
    g_ref[pl.ds(N, NP - N), :] = jnp.zeros((NP - N, D), jnp.float32)


def _dense1_body(q_ref, dinv_ref, b0_ref, w1_ref, g_ref):
    dinv = dinv_ref[...]
    qsum = q_ref[0, pl.ds(0, N), :] + q_ref[1, pl.ds(0, N), :]
    t = qsum * dinv + b0_ref[...]
    a = jnp.where(t > 0, t, 0.01 * t)
    h = jnp.dot(a, w1_ref[...], preferred_element_type=jnp.float32)
    g_ref[pl.ds(0, N), :] = h * dinv
    g_ref[pl.ds(N, NP - N), :] = jnp.zeros((NP - N, D), jnp.float32)


def _dense2_body(r_ref, dinv_ref, b1_ref, out_ref):
    rsum = r_ref[0, pl.ds(0, N), :] + r_ref[1, pl.ds(0, N), :]
    out_ref[...] = rsum * dinv_ref[...] + b1_ref[...]


@jax.jit
def kernel(x, edge_index, W0, b0, W1, b1):
    src = edge_index[0]
    dst = edge_index[1]

    zeros_blk = jnp.zeros((ZROWS, D), jnp.float32)

    degp = _deg_call(dst)

    h0 = pl.pallas_call(
        _mm0_body,
        out_shape=jax.ShapeDtypeStruct((N, D), jnp.float32),
    )(x, W0)

    g0, dinv = pl.pallas_call(
        _scale0_body,
        out_shape=[
            jax.ShapeDtypeStruct((NP, D), jnp.float32),
            jax.ShapeDtypeStruct((N, 1), jnp.float32),
        ],
    )(h0, degp)

    q = _agg_call(g0, src, dst, zeros_blk)

    g1 = pl.pallas_call(
        _dense1_body,
        out_shape=jax.ShapeDtypeStruct((NP, D), jnp.float32),
    )(q, dinv, b0.reshape(1, D), W1)

    r = _agg_call(g1, src, dst, zeros_blk)

    out = pl.pallas_call(
        _dense2_body,
        out_shape=jax.ShapeDtypeStruct((N, D), jnp.float32),
    )(r, dinv, b1.reshape(1, D))

    return out


# speedup vs baseline: 1.3469x; 1.0003x over previous
import jax
import jax.numpy as jnp
from jax import lax
from jax.experimental import pallas as pl
from jax.experimental.pallas import tpu as pltpu
from jax.experimental.pallas import tpu_sc as plsc

N = 10000
E = 320000
D = 128

NP = 10240
NC = 2
NS = 16
NW = NC * NS
EPW = E // NW
CH = 80
NCHUNK = EPW // CH
ROWS_PT = NP // NS
ZROWS = 32
DEG_W = 16

_mesh = plsc.VectorSubcoreMesh(core_axis_name="c", subcore_axis_name="s",
                               num_cores=NC, num_subcores=NS)


def _agg_body(g_hbm, src_hbm, dst_hbm, zeros_hbm, out_hbm,
              src_v, dst_v, rows_v, acc, gsem, ssem):
    c = lax.axis_index("c")
    s = lax.axis_index("s")
    wid = s * NC + c
    ebase = wid * EPW
    rbase = s * ROWS_PT

    @pl.when(c == 0)
    def _():
        pltpu.sync_copy(g_hbm.at[pl.ds(rbase, ROWS_PT)],
                        acc.at[pl.ds(rbase, ROWS_PT)])

    @pl.when(c != 0)
    def _():
        def zcopy(t, _):
            pltpu.sync_copy(zeros_hbm,
                            acc.at[pl.ds(rbase + t * ZROWS, ZROWS)])
            return 0

        lax.fori_loop(0, ROWS_PT // ZROWS, zcopy, 0)

    plsc.subcore_barrier()

    pltpu.sync_copy(src_hbm.at[pl.ds(ebase, CH)], src_v.at[0])
    pltpu.sync_copy(dst_hbm.at[pl.ds(ebase, CH)], dst_v.at[0])
    pltpu.async_copy(g_hbm.at[src_v.at[0]], rows_v.at[0], gsem)

    def chunk(j, _):
        p = lax.rem(j, 3)
        q = lax.rem(j + 1, 3)

        @pl.when(j >= 2)
        def _():
            pltpu.make_async_copy(rows_v.at[q], acc.at[dst_v.at[q]],
                                  ssem).wait()

        @pl.when(j < NCHUNK - 1)
        def _():
            pltpu.sync_copy(src_hbm.at[pl.ds(ebase + (j + 1) * CH, CH)],
                            src_v.at[q])
            pltpu.sync_copy(dst_hbm.at[pl.ds(ebase + (j + 1) * CH, CH)],
                            dst_v.at[q])
            pltpu.async_copy(g_hbm.at[src_v.at[q]], rows_v.at[q], gsem)

        pltpu.make_async_copy(g_hbm.at[src_v.at[p]], rows_v.at[p],
                              gsem).wait()
        pltpu.async_copy(rows_v.at[p], acc.at[dst_v.at[p]], ssem,
                         add=True)
        return 0

    lax.fori_loop(0, NCHUNK, chunk, 0)
    for t in (NCHUNK - 2, NCHUNK - 1):
        tp = t % 3
        pltpu.make_async_copy(rows_v.at[tp], acc.at[dst_v.at[tp]],
                              ssem).wait()

    plsc.subcore_barrier()
    pltpu.sync_copy(acc.at[pl.ds(rbase, ROWS_PT)],
                    out_hbm.at[c, pl.ds(rbase, ROWS_PT)])


_agg_call = pl.kernel(
    _agg_body,
    out_type=jax.ShapeDtypeStruct((NC, NP, D), jnp.float32),
    mesh=_mesh,
    scratch_types=[
        pltpu.VMEM((3, CH), jnp.int32),
        pltpu.VMEM((3, CH), jnp.int32),
        pltpu.VMEM((3, CH, D), jnp.float32),
        pltpu.VMEM_SHARED((NP, D), jnp.float32),
        pltpu.SemaphoreType.DMA,
        pltpu.SemaphoreType.DMA,
    ],
)


def _deg_body(dst_hbm, out_hbm, dst_v, ones_v, tcol_v, wide_v, acc, ssem):
    c = lax.axis_index("c")
    s = lax.axis_index("s")
    wid = s * NC + c
    ebase = wid * EPW
    rbase = s * ROWS_PT

    def fill_ones(i, _):
        ones_v[i, :] = jnp.ones((DEG_W,), jnp.float32)
        return 0

    lax.fori_loop(0, CH, fill_ones, 0)

    @pl.when(c == 0)
    def _():
        def fill_init(i, _):
            tcol_v[i, :] = jnp.ones((DEG_W,), jnp.float32)
            return 0
        lax.fori_loop(0, ZROWS, fill_init, 0)

    @pl.when(c != 0)
    def _():
        def fill_init(i, _):
            tcol_v[i, :] = jnp.zeros((DEG_W,), jnp.float32)
            return 0
        lax.fori_loop(0, ZROWS, fill_init, 0)

    def icopy(t, _):
        pltpu.sync_copy(tcol_v, acc.at[pl.ds(rbase + t * ZROWS, ZROWS)])
        return 0

    lax.fori_loop(0, ROWS_PT // ZROWS, icopy, 0)
    plsc.subcore_barrier()

    pltpu.sync_copy(dst_hbm.at[pl.ds(ebase, CH)], dst_v.at[0])

    def chunk(j, _):
        p = lax.rem(j, 2)
        q = lax.rem(j + 1, 2)

        @pl.when(j > 0)
        def _():
            pltpu.make_async_copy(ones_v, acc.at[dst_v.at[q]], ssem).wait()

        @pl.when(j < NCHUNK - 1)
        def _():
            pltpu.sync_copy(dst_hbm.at[pl.ds(ebase + (j + 1) * CH, CH)],
                            dst_v.at[q])

        pltpu.async_copy(ones_v, acc.at[dst_v.at[p]], ssem, add=True)
        return 0

    lax.fori_loop(0, NCHUNK, chunk, 0)
    lp = (NCHUNK - 1) % 2
    pltpu.make_async_copy(ones_v, acc.at[dst_v.at[lp]], ssem).wait()
    plsc.subcore_barrier()

    def wblock(t, _):
        pltpu.sync_copy(acc.at[pl.ds(rbase + t * ZROWS, ZROWS)], tcol_v)

        def repack(i, _):
            row = tcol_v[i, :]
            for k in range(D // DEG_W):
                wide_v[i, pl.ds(k * DEG_W, DEG_W)] = row
            return 0

        lax.fori_loop(0, ZROWS, repack, 0)
        pltpu.sync_copy(wide_v,
                        out_hbm.at[c, pl.ds(rbase + t * ZROWS, ZROWS)])
        return 0

    lax.fori_loop(0, ROWS_PT // ZROWS, wblock, 0)


_deg_call = pl.kernel(
    _deg_body,
    out_type=jax.ShapeDtypeStruct((NC, NP, D), jnp.float32),
    mesh=_mesh,
    scratch_types=[
        pltpu.VMEM((2, CH), jnp.int32),
        pltpu.VMEM((CH, DEG_W), jnp.float32),
        pltpu.VMEM((ZROWS, DEG_W), jnp.float32),
        pltpu.VMEM((ZROWS, D), jnp.float32),
        pltpu.VMEM_SHARED((NP, DEG_W), jnp.float32),
        pltpu.SemaphoreType.DMA,
    ],
    compiler_params=pltpu.CompilerParams(use_tc_tiling_on_sc=False),
)


def _mm0_body(x_ref, w_ref, h_ref):
    h_ref[...] = jnp.dot(x_ref[...], w_ref[...],
                         preferred_element_type=jnp.float32)


def _scale0_body(h_ref, degp_ref, g_ref, dinv_ref):
    d = degp_ref[0, pl.ds(0, N), :] + degp_ref[1, pl.ds(0, N), :]
    deg = jnp.max(d, axis=1, keepdims=True)
    dinv = lax.rsqrt(deg)
    dinv_ref[...] = dinv
    g_ref[pl.ds(0, N), :] = h_ref[...] * dinv
    g_ref[pl.ds(N, NP - N), :] = jnp.zeros((NP - N, D), jnp.float32)


def _dense1_body(q_ref, dinv_ref, b0_ref, w1_ref, g_ref):
    dinv = dinv_ref[...]
    qsum = q_ref[0, pl.ds(0, N), :] + q_ref[1, pl.ds(0, N), :]
    t = qsum * dinv + b0_ref[...]
    a = jnp.where(t > 0, t, 0.01 * t)
    h = jnp.dot(a, w1_ref[...], preferred_element_type=jnp.float32)
    g_ref[pl.ds(0, N), :] = h * dinv
    g_ref[pl.ds(N, NP - N), :] = jnp.zeros((NP - N, D), jnp.float32)


def _dense2_body(r_ref, dinv_ref, b1_ref, out_ref):
    rsum = r_ref[0, pl.ds(0, N), :] + r_ref[1, pl.ds(0, N), :]
    out_ref[...] = rsum * dinv_ref[...] + b1_ref[...]


@jax.jit
def kernel(x, edge_index, W0, b0, W1, b1):
    src = edge_index[0]
    dst = edge_index[1]

    zeros_blk = jnp.zeros((ZROWS, D), jnp.float32)

    degp = _deg_call(dst)

    h0 = pl.pallas_call(
        _mm0_body,
        out_shape=jax.ShapeDtypeStruct((N, D), jnp.float32),
    )(x, W0)

    g0, dinv = pl.pallas_call(
        _scale0_body,
        out_shape=[
            jax.ShapeDtypeStruct((NP, D), jnp.float32),
            jax.ShapeDtypeStruct((N, 1), jnp.float32),
        ],
    )(h0, degp)

    q = _agg_call(g0, src, dst, zeros_blk)

    g1 = pl.pallas_call(
        _dense1_body,
        out_shape=jax.ShapeDtypeStruct((NP, D), jnp.float32),
    )(q, dinv, b0.reshape(1, D), W1)

    r = _agg_call(g1, src, dst, zeros_blk)

    out = pl.pallas_call(
        _dense2_body,
        out_shape=jax.ShapeDtypeStruct((N, D), jnp.float32),
    )(r, dinv, b1.reshape(1, D))

    return out
